# baseline (speedup 1.0000x reference)
import functools

import jax
import jax.numpy as jnp
from jax import lax
from jax.experimental import pallas as pl
from jax.experimental.pallas import tpu as pltpu
from jax.experimental.pallas import tpu_sc as plsc

N = 10000
E = 320000
D = 128
R = 8
NCLS = 4

NC, NS, L = 2, 16, 16
NW = NC * NS
EPW = E // NW
CH = 80
NCHUNK = EPW // CH
NPAD = 10240
ROWS_PT = NPAD // NS

_SQRT2 = 1.4142135623730951


def _gelu(x):
    return 0.5 * x * (1.0 + lax.erf(x / _SQRT2))


def _mm_body(x_ref, wcat_ref, wroot_ref, y_ref, yr_ref):
    x = x_ref[...]
    y = jnp.dot(x, wcat_ref[...], preferred_element_type=jnp.float32)
    for r in range(R):
        y_ref[r] = y[:, r * D:(r + 1) * D]
    yr_ref[...] = jnp.dot(x, wroot_ref[...], preferred_element_type=jnp.float32)


def _mm(x, wcat, wroot):
    nb = 1000
    return pl.pallas_call(
        _mm_body,
        grid=(N // nb,),
        in_specs=[
            pl.BlockSpec((nb, D), lambda i: (i, 0)),
            pl.BlockSpec((D, R * D), lambda i: (0, 0)),
            pl.BlockSpec((D, D), lambda i: (0, 0)),
        ],
        out_specs=[
            pl.BlockSpec((R, nb, D), lambda i: (0, i, 0)),
            pl.BlockSpec((nb, D), lambda i: (i, 0)),
        ],
        out_shape=[
            jax.ShapeDtypeStruct((R, N, D), jnp.float32),
            jax.ShapeDtypeStruct((N, D), jnp.float32),
        ],
    )(x, wcat, wroot)


def _ln_relu(acc0, acc1, deg, xr, b, g, lb):
    inv = 1.0 / jnp.maximum(deg, 1.0)
    h = (acc0 + acc1) * inv + xr + b
    h = jnp.maximum(h, 0.0)
    m = jnp.mean(h, axis=-1, keepdims=True)
    v = jnp.mean((h - m) * (h - m), axis=-1, keepdims=True)
    return (h - m) * lax.rsqrt(v + 1e-5) * g + lb


def _norm1mm_body(acc_ref, dacc_ref, xr_ref, b_ref, g_ref, lb_ref,
                  wcat_ref, wroot_ref, h1_ref, y2_ref, xr2_ref):
    deg = dacc_ref[0, :, 0:1] + dacc_ref[1, :, 0:1]
    h1 = _ln_relu(acc_ref[0], acc_ref[1], deg, xr_ref[...],
                  b_ref[...], g_ref[...], lb_ref[...])
    h1_ref[...] = h1
    y2 = jnp.dot(h1, wcat_ref[...], preferred_element_type=jnp.float32)
    for r in range(R):
        y2_ref[r] = y2[:, r * D:(r + 1) * D]
    xr2_ref[...] = jnp.dot(h1, wroot_ref[...],
                           preferred_element_type=jnp.float32)


def _norm1mm(acc, dacc, xr, b, g, lb, wcat, wroot):
    nb = 1000
    return pl.pallas_call(
        _norm1mm_body,
        grid=(N // nb,),
        in_specs=[
            pl.BlockSpec((NC, nb, D), lambda i: (0, i, 0)),
            pl.BlockSpec((NC, nb, D), lambda i: (0, i, 0)),
            pl.BlockSpec((nb, D), lambda i: (i, 0)),
            pl.BlockSpec((1, D), lambda i: (0, 0)),
            pl.BlockSpec((1, D), lambda i: (0, 0)),
            pl.BlockSpec((1, D), lambda i: (0, 0)),
            pl.BlockSpec((D, R * D), lambda i: (0, 0)),
            pl.BlockSpec((D, D), lambda i: (0, 0)),
        ],
        out_specs=[
            pl.BlockSpec((nb, D), lambda i: (i, 0)),
            pl.BlockSpec((R, nb, D), lambda i: (0, i, 0)),
            pl.BlockSpec((nb, D), lambda i: (i, 0)),
        ],
        out_shape=[
            jax.ShapeDtypeStruct((N, D), jnp.float32),
            jax.ShapeDtypeStruct((R, N, D), jnp.float32),
            jax.ShapeDtypeStruct((N, D), jnp.float32),
        ],
    )(acc, dacc, xr, b, g, lb, wcat, wroot)


def _norm2_body(acc_ref, dacc_ref, xr_ref, b_ref, g_ref, lb_ref, h1_ref,
                e1a_ref, e1bw_ref, e1b_ref, a_ref, bb_ref):
    deg = dacc_ref[0, :, 0:1] + dacc_ref[1, :, 0:1]
    h2 = _ln_relu(acc_ref[0], acc_ref[1], deg, xr_ref[...],
                  b_ref[...], g_ref[...], lb_ref[...])
    h = h1_ref[...] + h2
    a_ref[...] = (jnp.dot(h, e1a_ref[...], preferred_element_type=jnp.float32)
                  + e1b_ref[...])
    bb_ref[...] = jnp.dot(h, e1bw_ref[...], preferred_element_type=jnp.float32)


def _norm2(acc, dacc, xr, b, g, lb, h1, e1a, e1bw, e1b):
    nb = 1000
    return pl.pallas_call(
        _norm2_body,
        grid=(N // nb,),
        in_specs=[
            pl.BlockSpec((NC, nb, D), lambda i: (0, i, 0)),
            pl.BlockSpec((NC, nb, D), lambda i: (0, i, 0)),
            pl.BlockSpec((nb, D), lambda i: (i, 0)),
            pl.BlockSpec((1, D), lambda i: (0, 0)),
            pl.BlockSpec((1, D), lambda i: (0, 0)),
            pl.BlockSpec((1, D), lambda i: (0, 0)),
            pl.BlockSpec((nb, D), lambda i: (i, 0)),
            pl.BlockSpec((D, D), lambda i: (0, 0)),
            pl.BlockSpec((D, D), lambda i: (0, 0)),
            pl.BlockSpec((1, D), lambda i: (0, 0)),
        ],
        out_specs=[
            pl.BlockSpec((nb, D), lambda i: (i, 0)),
            pl.BlockSpec((nb, D), lambda i: (i, 0)),
        ],
        out_shape=[
            jax.ShapeDtypeStruct((N, D), jnp.float32),
            jax.ShapeDtypeStruct((N, D), jnp.float32),
        ],
    )(acc, dacc, xr, b, g, lb, h1, e1a, e1bw, e1b)


def _dec_body(z_ref, w2_ref, b2_ref, w3_ref, b3_ref, out_ref):
    z = _gelu(z_ref[...])
    t = _gelu(jnp.dot(z, w2_ref[...], preferred_element_type=jnp.float32)
              + b2_ref[...])
    out_ref[...] = (jnp.dot(t, w3_ref[...], preferred_element_type=jnp.float32)
                    + b3_ref[...])


def _dec_mlp(z1, w2, b2, w3, b3):
    nb = 6400
    ne = z1.shape[0]
    return pl.pallas_call(
        _dec_body,
        grid=(ne // nb,),
        in_specs=[
            pl.BlockSpec((nb, D), lambda i: (i, 0)),
            pl.BlockSpec((D, D // 2), lambda i: (0, 0)),
            pl.BlockSpec((1, D // 2), lambda i: (0, 0)),
            pl.BlockSpec((D // 2, NCLS), lambda i: (0, 0)),
            pl.BlockSpec((1, NCLS), lambda i: (0, 0)),
        ],
        out_specs=pl.BlockSpec((nb, NCLS), lambda i: (i, 0)),
        out_shape=jax.ShapeDtypeStruct((ne, NCLS), jnp.float32),
    )(z1, w2, b2, w3, b3)


_MESH = plsc.VectorSubcoreMesh(core_axis_name="c", subcore_axis_name="s")


SCN = 5
SCR = NCHUNK // SCN


def _conv_sc_body(table, src4, et4, dst4, acc_out,
                  src_b, et_b, dst_b, flat, gbuf, acc_sh, sem0, sem1):
    c = lax.axis_index("c")
    s = lax.axis_index("s")
    wid = s * NC + c
    zeros16 = jnp.zeros((L,), jnp.float32)

    def _zrow(i, _):
        for j in range(D // L):
            gbuf[0, i, pl.ds(j * L, L)] = zeros16
        return 0
    lax.fori_loop(0, CH, _zrow, 0)

    for k in range(ROWS_PT // CH):
        row0 = s * ROWS_PT + k * CH
        pltpu.sync_copy(gbuf.at[0], acc_sh.at[pl.ds(row0, CH)])

    plsc.subcore_barrier()

    def _flatidx(p, j):
        for t in range(CH // L):
            sl = pl.ds(t * L, L)
            flat[p, sl] = et_b[j, sl] * N + src_b[j, sl]

    def _super(sc, _):
        pltpu.sync_copy(src4.at[wid, sc], src_b)
        pltpu.sync_copy(et4.at[wid, sc], et_b)
        pltpu.sync_copy(dst4.at[wid, sc], dst_b)

        _flatidx(0, 0)
        pltpu.async_copy(table.at[flat.at[0]], gbuf.at[0], sem0)

        def _pair(k, _):
            j0 = 2 * k + 1
            j1 = 2 * k + 2
            _flatidx(1, j0)
            pltpu.async_copy(table.at[flat.at[1]], gbuf.at[1], sem1)
            pltpu.make_async_copy(table.at[flat.at[0]], gbuf.at[0], sem0).wait()
            pltpu.sync_copy(gbuf.at[0], acc_sh.at[dst_b.at[2 * k]], add=True)
            _flatidx(0, j1)
            pltpu.async_copy(table.at[flat.at[0]], gbuf.at[0], sem0)
            pltpu.make_async_copy(table.at[flat.at[1]], gbuf.at[1], sem1).wait()
            pltpu.sync_copy(gbuf.at[1], acc_sh.at[dst_b.at[j0]], add=True)
            return 0
        lax.fori_loop(0, (SCR - 1) // 2, _pair, 0)

        pltpu.make_async_copy(table.at[flat.at[0]], gbuf.at[0], sem0).wait()
        pltpu.sync_copy(gbuf.at[0], acc_sh.at[dst_b.at[SCR - 1]], add=True)
        return 0
    lax.fori_loop(0, SCN, _super, 0)

    plsc.subcore_barrier()

    for k in range(ROWS_PT // CH):
        row0 = s * ROWS_PT + k * CH
        pltpu.sync_copy(acc_sh.at[pl.ds(row0, CH)], gbuf.at[0])
        pltpu.sync_copy(gbuf.at[0], acc_out.at[c, pl.ds(row0, CH)])


def _deg_sc_body(dst4, deg_out, dst_b, gbuf, acc_sh, sem):
    c = lax.axis_index("c")
    s = lax.axis_index("s")
    wid = s * NC + c
    zeros16 = jnp.zeros((L,), jnp.float32)
    ones16 = jnp.ones((L,), jnp.float32)

    def _zrow(i, _):
        for j in range(D // L):
            gbuf[i, pl.ds(j * L, L)] = zeros16
        return 0
    lax.fori_loop(0, CH, _zrow, 0)

    for k in range(ROWS_PT // CH):
        row0 = s * ROWS_PT + k * CH
        pltpu.sync_copy(gbuf, acc_sh.at[pl.ds(row0, CH)])

    def _orow(i, _):
        for j in range(D // L):
            gbuf[i, pl.ds(j * L, L)] = ones16
        return 0
    lax.fori_loop(0, CH, _orow, 0)

    pltpu.sync_copy(dst4.at[wid], dst_b)
    plsc.subcore_barrier()

    def _chunk(i, _):
        pltpu.sync_copy(gbuf, acc_sh.at[dst_b.at[i]], add=True)
        return 0
    lax.fori_loop(0, NCHUNK, _chunk, 0)

    plsc.subcore_barrier()

    for k in range(ROWS_PT // CH):
        row0 = s * ROWS_PT + k * CH
        pltpu.sync_copy(acc_sh.at[pl.ds(row0, CH)], gbuf)
        pltpu.sync_copy(gbuf, deg_out.at[c, pl.ds(row0, CH)])


_conv = pl.kernel(
    _conv_sc_body,
    out_type=jax.ShapeDtypeStruct((NC, NPAD, D), jnp.float32),
    mesh=_MESH,
    scratch_types=[
        pltpu.VMEM((SCR, CH), jnp.int32),
        pltpu.VMEM((SCR, CH), jnp.int32),
        pltpu.VMEM((SCR, CH), jnp.int32),
        pltpu.VMEM((2, CH), jnp.int32),
        pltpu.VMEM((2, CH, D), jnp.float32),
        pltpu.VMEM_SHARED((NPAD, D), jnp.float32),
        pltpu.SemaphoreType.DMA,
        pltpu.SemaphoreType.DMA,
    ],
)

_deg = pl.kernel(
    _deg_sc_body,
    out_type=jax.ShapeDtypeStruct((NC, NPAD, D), jnp.float32),
    mesh=_MESH,
    scratch_types=[
        pltpu.VMEM((NCHUNK, CH), jnp.int32),
        pltpu.VMEM((CH, D), jnp.float32),
        pltpu.VMEM_SHARED((NPAD, D), jnp.float32),
        pltpu.SemaphoreType.DMA,
    ],
)


def _decgather_body(nch, a, bb, sidx4, didx4, z1,
                    si_b, di_b, ga, gb, sem0, sem1):
    c = lax.axis_index("c")
    s = lax.axis_index("s")
    wid = s * NC + c
    base = wid * (nch * CH)

    pltpu.sync_copy(sidx4.at[wid], si_b)
    pltpu.sync_copy(didx4.at[wid], di_b)

    def _gath(j, p, sem):
        pltpu.async_copy(a.at[si_b.at[j]], ga.at[p], sem)
        pltpu.async_copy(bb.at[di_b.at[j]], gb.at[p], sem)

    def _waitg(j, p, sem):
        pltpu.make_async_copy(a.at[si_b.at[j]], ga.at[p], sem).wait()
        pltpu.make_async_copy(bb.at[di_b.at[j]], gb.at[p], sem).wait()

    def _addwrite(j, p):
        def _row(r, _):
            for t in range(D // L):
                sl = pl.ds(t * L, L)
                plsc.addupdate(ga.at[p, r, sl], gb[p, r, sl])
            return 0
        lax.fori_loop(0, CH, _row, 0)
        pltpu.sync_copy(ga.at[p], z1.at[pl.ds(base + j * CH, CH)])

    _gath(0, 0, sem0)

    def _pair(k, _):
        j0 = 2 * k
        j1 = 2 * k + 1
        j2 = 2 * k + 2
        _gath(j1, 1, sem1)
        _waitg(j0, 0, sem0)
        _addwrite(j0, 0)
        _gath(j2, 0, sem0)
        _waitg(j1, 1, sem1)
        _addwrite(j1, 1)
        return 0
    lax.fori_loop(0, (nch - 1) // 2, _pair, 0)

    if nch % 2 == 1:
        _waitg(nch - 1, 0, sem0)
        _addwrite(nch - 1, 0)
    else:
        _gath(nch - 1, 1, sem1)
        _waitg(nch - 2, 0, sem0)
        _addwrite(nch - 2, 0)
        _waitg(nch - 1, 1, sem1)
        _addwrite(nch - 1, 1)


def _make_decgather(ne):
    nch = ne // (NW * CH)
    return pl.kernel(
        functools.partial(_decgather_body, nch),
        out_type=jax.ShapeDtypeStruct((ne, D), jnp.float32),
        mesh=_MESH,
        scratch_types=[
            pltpu.VMEM((nch, CH), jnp.int32),
            pltpu.VMEM((nch, CH), jnp.int32),
            pltpu.VMEM((2, CH, D), jnp.float32),
            pltpu.VMEM((2, CH, D), jnp.float32),
            pltpu.SemaphoreType.DMA,
            pltpu.SemaphoreType.DMA,
        ],
    )


_DEC_SPLITS = (128000, 128000, 64000)
_decgather_k = {ne: _make_decgather(ne) for ne in set(_DEC_SPLITS)}


@jax.jit
def kernel(x, edge_index, edge_type, edges,
           W1, W1_root, b1, W2, W2_root, b2,
           ln1_g, ln1_b, ln2_g, ln2_b,
           e1_w, e1_b, e2_w, e2_b, e3_w, e3_b):
    src = edge_index[0].astype(jnp.int32)
    dst = edge_index[1].astype(jnp.int32)
    et = edge_type.astype(jnp.int32)
    sidx = edges[:, 0].astype(jnp.int32)
    didx = edges[:, 1].astype(jnp.int32)

    w1cat = jnp.transpose(W1, (1, 0, 2)).reshape(D, R * D)
    w2cat = jnp.transpose(W2, (1, 0, 2)).reshape(D, R * D)

    y1, xr1 = _mm(x, w1cat, W1_root)
    src4 = src.reshape(NW, SCN, SCR, CH)
    et4 = et.reshape(NW, SCN, SCR, CH)
    dst4 = dst.reshape(NW, SCN, SCR, CH)
    acc1 = _conv(y1.reshape(N * R, D), src4, et4, dst4)
    dacc = _deg(dst.reshape(NW, NCHUNK, CH))
    h1, y2, xr2 = _norm1mm(acc1, dacc, xr1, b1.reshape(1, D),
                           ln1_g.reshape(1, D), ln1_b.reshape(1, D),
                           w2cat, W2_root)

    acc2 = _conv(y2.reshape(N * R, D), src4, et4, dst4)
    a, bb = _norm2(acc2, dacc, xr2, b2.reshape(1, D),
                   ln2_g.reshape(1, D), ln2_b.reshape(1, D),
                   h1, e1_w[:D], e1_w[D:], e1_b.reshape(1, D))

    outs = []
    off = 0
    for ne in _DEC_SPLITS:
        nch = ne // (NW * CH)
        z1p = _decgather_k[ne](a, bb,
                               sidx[off:off + ne].reshape(NW, nch, CH),
                               didx[off:off + ne].reshape(NW, nch, CH))
        outs.append(_dec_mlp(z1p, e2_w, e2_b.reshape(1, D // 2),
                             e3_w, e3_b.reshape(1, NCLS)))
        off += ne
    return jnp.concatenate(outs, axis=0)


# speedup vs baseline: 1.0239x; 1.0239x over previous
import functools

import jax
import jax.numpy as jnp
from jax import lax
from jax.experimental import pallas as pl
from jax.experimental.pallas import tpu as pltpu
from jax.experimental.pallas import tpu_sc as plsc

N = 10000
E = 320000
D = 128
R = 8
NCLS = 4

NC, NS, L = 2, 16, 16
NW = NC * NS
EPW = E // NW
CH = 80
NCHUNK = EPW // CH
NPAD = 10240
ROWS_PT = NPAD // NS

_SQRT2 = 1.4142135623730951


def _gelu(x):
    return 0.5 * x * (1.0 + lax.erf(x / _SQRT2))


def _mm_body(x_ref, wcat_ref, wroot_ref, y_ref, yr_ref):
    x = x_ref[...]
    y = jnp.dot(x, wcat_ref[...], preferred_element_type=jnp.float32)
    for r in range(R):
        y_ref[r] = y[:, r * D:(r + 1) * D]
    yr_ref[...] = jnp.dot(x, wroot_ref[...], preferred_element_type=jnp.float32)


def _mm(x, wcat, wroot):
    nb = 1000
    return pl.pallas_call(
        _mm_body,
        grid=(N // nb,),
        in_specs=[
            pl.BlockSpec((nb, D), lambda i: (i, 0)),
            pl.BlockSpec((D, R * D), lambda i: (0, 0)),
            pl.BlockSpec((D, D), lambda i: (0, 0)),
        ],
        out_specs=[
            pl.BlockSpec((R, nb, D), lambda i: (0, i, 0)),
            pl.BlockSpec((nb, D), lambda i: (i, 0)),
        ],
        out_shape=[
            jax.ShapeDtypeStruct((R, N, D), jnp.float32),
            jax.ShapeDtypeStruct((N, D), jnp.float32),
        ],
    )(x, wcat, wroot)


def _ln_relu(acc0, acc1, deg, xr, b, g, lb):
    inv = 1.0 / jnp.maximum(deg, 1.0)
    h = (acc0 + acc1) * inv + xr + b
    h = jnp.maximum(h, 0.0)
    m = jnp.mean(h, axis=-1, keepdims=True)
    v = jnp.mean((h - m) * (h - m), axis=-1, keepdims=True)
    return (h - m) * lax.rsqrt(v + 1e-5) * g + lb


def _norm1mm_body(acc_ref, dacc_ref, xr_ref, b_ref, g_ref, lb_ref,
                  wcat_ref, wroot_ref, h1_ref, y2_ref, xr2_ref):
    deg = dacc_ref[0, :, 0:1] + dacc_ref[1, :, 0:1]
    h1 = _ln_relu(acc_ref[0], acc_ref[1], deg, xr_ref[...],
                  b_ref[...], g_ref[...], lb_ref[...])
    h1_ref[...] = h1
    y2 = jnp.dot(h1, wcat_ref[...], preferred_element_type=jnp.float32)
    for r in range(R):
        y2_ref[r] = y2[:, r * D:(r + 1) * D]
    xr2_ref[...] = jnp.dot(h1, wroot_ref[...],
                           preferred_element_type=jnp.float32)


def _norm1mm(acc, dacc, xr, b, g, lb, wcat, wroot):
    nb = 1000
    return pl.pallas_call(
        _norm1mm_body,
        grid=(N // nb,),
        in_specs=[
            pl.BlockSpec((NC, nb, D), lambda i: (0, i, 0)),
            pl.BlockSpec((NC, nb, D), lambda i: (0, i, 0)),
            pl.BlockSpec((nb, D), lambda i: (i, 0)),
            pl.BlockSpec((1, D), lambda i: (0, 0)),
            pl.BlockSpec((1, D), lambda i: (0, 0)),
            pl.BlockSpec((1, D), lambda i: (0, 0)),
            pl.BlockSpec((D, R * D), lambda i: (0, 0)),
            pl.BlockSpec((D, D), lambda i: (0, 0)),
        ],
        out_specs=[
            pl.BlockSpec((nb, D), lambda i: (i, 0)),
            pl.BlockSpec((R, nb, D), lambda i: (0, i, 0)),
            pl.BlockSpec((nb, D), lambda i: (i, 0)),
        ],
        out_shape=[
            jax.ShapeDtypeStruct((N, D), jnp.float32),
            jax.ShapeDtypeStruct((R, N, D), jnp.float32),
            jax.ShapeDtypeStruct((N, D), jnp.float32),
        ],
    )(acc, dacc, xr, b, g, lb, wcat, wroot)


def _norm2_body(acc_ref, dacc_ref, xr_ref, b_ref, g_ref, lb_ref, h1_ref,
                e1a_ref, e1bw_ref, e1b_ref, a_ref, bb_ref):
    deg = dacc_ref[0, :, 0:1] + dacc_ref[1, :, 0:1]
    h2 = _ln_relu(acc_ref[0], acc_ref[1], deg, xr_ref[...],
                  b_ref[...], g_ref[...], lb_ref[...])
    h = h1_ref[...] + h2
    a_ref[...] = (jnp.dot(h, e1a_ref[...], preferred_element_type=jnp.float32)
                  + e1b_ref[...])
    bb_ref[...] = jnp.dot(h, e1bw_ref[...], preferred_element_type=jnp.float32)


def _norm2(acc, dacc, xr, b, g, lb, h1, e1a, e1bw, e1b):
    nb = 1000
    return pl.pallas_call(
        _norm2_body,
        grid=(N // nb,),
        in_specs=[
            pl.BlockSpec((NC, nb, D), lambda i: (0, i, 0)),
            pl.BlockSpec((NC, nb, D), lambda i: (0, i, 0)),
            pl.BlockSpec((nb, D), lambda i: (i, 0)),
            pl.BlockSpec((1, D), lambda i: (0, 0)),
            pl.BlockSpec((1, D), lambda i: (0, 0)),
            pl.BlockSpec((1, D), lambda i: (0, 0)),
            pl.BlockSpec((nb, D), lambda i: (i, 0)),
            pl.BlockSpec((D, D), lambda i: (0, 0)),
            pl.BlockSpec((D, D), lambda i: (0, 0)),
            pl.BlockSpec((1, D), lambda i: (0, 0)),
        ],
        out_specs=[
            pl.BlockSpec((nb, D), lambda i: (i, 0)),
            pl.BlockSpec((nb, D), lambda i: (i, 0)),
        ],
        out_shape=[
            jax.ShapeDtypeStruct((N, D), jnp.float32),
            jax.ShapeDtypeStruct((N, D), jnp.float32),
        ],
    )(acc, dacc, xr, b, g, lb, h1, e1a, e1bw, e1b)


def _dec_body(z_ref, w2_ref, b2_ref, w3_ref, b3_ref, out_ref):
    z = _gelu(z_ref[...])
    t = _gelu(jnp.dot(z, w2_ref[...], preferred_element_type=jnp.float32)
              + b2_ref[...])
    out_ref[...] = (jnp.dot(t, w3_ref[...], preferred_element_type=jnp.float32)
                    + b3_ref[...])


def _dec_mlp(z1, w2, b2, w3, b3):
    nb = 6400
    ne = z1.shape[0]
    return pl.pallas_call(
        _dec_body,
        grid=(ne // nb,),
        in_specs=[
            pl.BlockSpec((nb, D), lambda i: (i, 0)),
            pl.BlockSpec((D, D // 2), lambda i: (0, 0)),
            pl.BlockSpec((1, D // 2), lambda i: (0, 0)),
            pl.BlockSpec((D // 2, NCLS), lambda i: (0, 0)),
            pl.BlockSpec((1, NCLS), lambda i: (0, 0)),
        ],
        out_specs=pl.BlockSpec((nb, NCLS), lambda i: (i, 0)),
        out_shape=jax.ShapeDtypeStruct((ne, NCLS), jnp.float32),
    )(z1, w2, b2, w3, b3)


_MESH = plsc.VectorSubcoreMesh(core_axis_name="c", subcore_axis_name="s")


SCN = 5
SCR = NCHUNK // SCN


def _conv_sc_body(table, src4, et4, dst4, acc_out,
                  src_b, et_b, dst_b, flat, gbuf, acc_sh, sem0, sem1):
    c = lax.axis_index("c")
    s = lax.axis_index("s")
    wid = s * NC + c
    zeros16 = jnp.zeros((L,), jnp.float32)

    def _zrow(i, _):
        for j in range(D // L):
            gbuf[0, i, pl.ds(j * L, L)] = zeros16
        return 0
    lax.fori_loop(0, CH, _zrow, 0)

    for k in range(ROWS_PT // CH):
        row0 = s * ROWS_PT + k * CH
        pltpu.sync_copy(gbuf.at[0], acc_sh.at[pl.ds(row0, CH)])

    plsc.subcore_barrier()

    def _flatidx(p, j):
        for t in range(CH // L):
            sl = pl.ds(t * L, L)
            flat[p, sl] = et_b[j, sl] * N + src_b[j, sl]

    def _super(sc, _):
        pltpu.sync_copy(src4.at[wid, sc], src_b)
        pltpu.sync_copy(et4.at[wid, sc], et_b)
        pltpu.sync_copy(dst4.at[wid, sc], dst_b)

        _flatidx(0, 0)
        pltpu.async_copy(table.at[flat.at[0]], gbuf.at[0], sem0)

        def _pair(k, _):
            j0 = 2 * k + 1
            j1 = 2 * k + 2
            _flatidx(1, j0)
            pltpu.async_copy(table.at[flat.at[1]], gbuf.at[1], sem1)
            pltpu.make_async_copy(table.at[flat.at[0]], gbuf.at[0], sem0).wait()
            pltpu.sync_copy(gbuf.at[0], acc_sh.at[dst_b.at[2 * k]], add=True)
            _flatidx(0, j1)
            pltpu.async_copy(table.at[flat.at[0]], gbuf.at[0], sem0)
            pltpu.make_async_copy(table.at[flat.at[1]], gbuf.at[1], sem1).wait()
            pltpu.sync_copy(gbuf.at[1], acc_sh.at[dst_b.at[j0]], add=True)
            return 0
        lax.fori_loop(0, (SCR - 1) // 2, _pair, 0)

        pltpu.make_async_copy(table.at[flat.at[0]], gbuf.at[0], sem0).wait()
        pltpu.sync_copy(gbuf.at[0], acc_sh.at[dst_b.at[SCR - 1]], add=True)
        return 0
    lax.fori_loop(0, SCN, _super, 0)

    plsc.subcore_barrier()

    for k in range(ROWS_PT // CH):
        row0 = s * ROWS_PT + k * CH
        pltpu.sync_copy(acc_sh.at[pl.ds(row0, CH)], gbuf.at[0])
        pltpu.sync_copy(gbuf.at[0], acc_out.at[c, pl.ds(row0, CH)])


def _deg_sc_body(dst4, deg_out, dst_b, gbuf, acc_sh, sem):
    c = lax.axis_index("c")
    s = lax.axis_index("s")
    wid = s * NC + c
    zeros16 = jnp.zeros((L,), jnp.float32)
    ones16 = jnp.ones((L,), jnp.float32)

    def _zrow(i, _):
        for j in range(D // L):
            gbuf[i, pl.ds(j * L, L)] = zeros16
        return 0
    lax.fori_loop(0, CH, _zrow, 0)

    for k in range(ROWS_PT // CH):
        row0 = s * ROWS_PT + k * CH
        pltpu.sync_copy(gbuf, acc_sh.at[pl.ds(row0, CH)])

    def _orow(i, _):
        for j in range(D // L):
            gbuf[i, pl.ds(j * L, L)] = ones16
        return 0
    lax.fori_loop(0, CH, _orow, 0)

    pltpu.sync_copy(dst4.at[wid], dst_b)
    plsc.subcore_barrier()

    def _chunk(i, _):
        pltpu.sync_copy(gbuf, acc_sh.at[dst_b.at[i]], add=True)
        return 0
    lax.fori_loop(0, NCHUNK, _chunk, 0)

    plsc.subcore_barrier()

    for k in range(ROWS_PT // CH):
        row0 = s * ROWS_PT + k * CH
        pltpu.sync_copy(acc_sh.at[pl.ds(row0, CH)], gbuf)
        pltpu.sync_copy(gbuf, deg_out.at[c, pl.ds(row0, CH)])


_conv = pl.kernel(
    _conv_sc_body,
    out_type=jax.ShapeDtypeStruct((NC, NPAD, D), jnp.float32),
    mesh=_MESH,
    scratch_types=[
        pltpu.VMEM((SCR, CH), jnp.int32),
        pltpu.VMEM((SCR, CH), jnp.int32),
        pltpu.VMEM((SCR, CH), jnp.int32),
        pltpu.VMEM((2, CH), jnp.int32),
        pltpu.VMEM((2, CH, D), jnp.float32),
        pltpu.VMEM_SHARED((NPAD, D), jnp.float32),
        pltpu.SemaphoreType.DMA,
        pltpu.SemaphoreType.DMA,
    ],
)

_deg = pl.kernel(
    _deg_sc_body,
    out_type=jax.ShapeDtypeStruct((NC, NPAD, D), jnp.float32),
    mesh=_MESH,
    scratch_types=[
        pltpu.VMEM((NCHUNK, CH), jnp.int32),
        pltpu.VMEM((CH, D), jnp.float32),
        pltpu.VMEM_SHARED((NPAD, D), jnp.float32),
        pltpu.SemaphoreType.DMA,
    ],
)


def _decgather_body(nch, a, bb, sidx4, didx4, z1,
                    si_b, di_b, ga, gb, sem0, sem1):
    c = lax.axis_index("c")
    s = lax.axis_index("s")
    wid = s * NC + c
    base = wid * (nch * CH)

    pltpu.sync_copy(sidx4.at[wid], si_b)
    pltpu.sync_copy(didx4.at[wid], di_b)

    def _gath(j, p, sem):
        pltpu.async_copy(a.at[si_b.at[j]], ga.at[p], sem)
        pltpu.async_copy(bb.at[di_b.at[j]], gb.at[p], sem)

    def _waitg(j, p, sem):
        pltpu.make_async_copy(a.at[si_b.at[j]], ga.at[p], sem).wait()
        pltpu.make_async_copy(bb.at[di_b.at[j]], gb.at[p], sem).wait()

    def _addwrite(j, p):
        def _row(r, _):
            for t in range(D // L):
                sl = pl.ds(t * L, L)
                plsc.addupdate(ga.at[p, r, sl], gb[p, r, sl])
            return 0
        lax.fori_loop(0, CH, _row, 0)
        pltpu.sync_copy(ga.at[p], z1.at[pl.ds(base + j * CH, CH)])

    _gath(0, 0, sem0)

    def _pair(k, _):
        j0 = 2 * k
        j1 = 2 * k + 1
        j2 = 2 * k + 2
        _gath(j1, 1, sem1)
        _waitg(j0, 0, sem0)
        _addwrite(j0, 0)
        _gath(j2, 0, sem0)
        _waitg(j1, 1, sem1)
        _addwrite(j1, 1)
        return 0
    lax.fori_loop(0, (nch - 1) // 2, _pair, 0)

    if nch % 2 == 1:
        _waitg(nch - 1, 0, sem0)
        _addwrite(nch - 1, 0)
    else:
        _gath(nch - 1, 1, sem1)
        _waitg(nch - 2, 0, sem0)
        _addwrite(nch - 2, 0)
        _waitg(nch - 1, 1, sem1)
        _addwrite(nch - 1, 1)


def _make_decgather(ne):
    nch = ne // (NW * CH)
    return pl.kernel(
        functools.partial(_decgather_body, nch),
        out_type=jax.ShapeDtypeStruct((ne, D), jnp.float32),
        mesh=_MESH,
        scratch_types=[
            pltpu.VMEM((nch, CH), jnp.int32),
            pltpu.VMEM((nch, CH), jnp.int32),
            pltpu.VMEM((2, CH, D), jnp.float32),
            pltpu.VMEM((2, CH, D), jnp.float32),
            pltpu.SemaphoreType.DMA,
            pltpu.SemaphoreType.DMA,
        ],
    )


EA = 192000
_decgather_a = _make_decgather(EA)
_decgather_b = _make_decgather(E - EA)


@jax.jit
def kernel(x, edge_index, edge_type, edges,
           W1, W1_root, b1, W2, W2_root, b2,
           ln1_g, ln1_b, ln2_g, ln2_b,
           e1_w, e1_b, e2_w, e2_b, e3_w, e3_b):
    src = edge_index[0].astype(jnp.int32)
    dst = edge_index[1].astype(jnp.int32)
    et = edge_type.astype(jnp.int32)
    sidx = edges[:, 0].astype(jnp.int32)
    didx = edges[:, 1].astype(jnp.int32)

    w1cat = jnp.transpose(W1, (1, 0, 2)).reshape(D, R * D)
    w2cat = jnp.transpose(W2, (1, 0, 2)).reshape(D, R * D)

    y1, xr1 = _mm(x, w1cat, W1_root)
    src4 = src.reshape(NW, SCN, SCR, CH)
    et4 = et.reshape(NW, SCN, SCR, CH)
    dst4 = dst.reshape(NW, SCN, SCR, CH)
    acc1 = _conv(y1.reshape(N * R, D), src4, et4, dst4)
    dacc = _deg(dst.reshape(NW, NCHUNK, CH))
    h1, y2, xr2 = _norm1mm(acc1, dacc, xr1, b1.reshape(1, D),
                           ln1_g.reshape(1, D), ln1_b.reshape(1, D),
                           w2cat, W2_root)

    acc2 = _conv(y2.reshape(N * R, D), src4, et4, dst4)
    a, bb = _norm2(acc2, dacc, xr2, b2.reshape(1, D),
                   ln2_g.reshape(1, D), ln2_b.reshape(1, D),
                   h1, e1_w[:D], e1_w[D:], e1_b.reshape(1, D))

    ncha = EA // (NW * CH)
    nchb = (E - EA) // (NW * CH)
    z1a = _decgather_a(a, bb, sidx[:EA].reshape(NW, ncha, CH),
                       didx[:EA].reshape(NW, ncha, CH))
    z1b = _decgather_b(a, bb, sidx[EA:].reshape(NW, nchb, CH),
                       didx[EA:].reshape(NW, nchb, CH))
    outa = _dec_mlp(z1a, e2_w, e2_b.reshape(1, D // 2),
                    e3_w, e3_b.reshape(1, NCLS))
    outb = _dec_mlp(z1b, e2_w, e2_b.reshape(1, D // 2),
                    e3_w, e3_b.reshape(1, NCLS))
    return jnp.concatenate([outa, outb], axis=0)
